# Initial kernel scaffold; baseline (speedup 1.0000x reference)
import jax
import jax.numpy as jnp
from jax.experimental import pallas as pl


def kernel(x, scales):
    raise NotImplementedError("write your pallas kernel here")


# speedup vs baseline: 2.9060x; 2.9060x over previous
import functools

import jax
import jax.numpy as jnp
from jax import lax
from jax.experimental import pallas as pl
from jax.experimental.pallas import tpu as pltpu
from jax.experimental.pallas import tpu_sc as plsc

NC = 2
NS = 16
NW = NC * NS
L = 16

N = 2048 * 2048
PER_W = N // NW
CHUNK = 16384
NCH = PER_W // CHUNK
U = 8
VECS = CHUNK // L

_f32 = jnp.float32


def _worker_id():
    return lax.axis_index("c") * NS + lax.axis_index("s")


def _streaming_body(compute_chunk, x_hbm, out_hbm, xb, ob, si, so):
    wid = _worker_id()
    base = wid * PER_W

    def start_in(c):
        return pltpu.async_copy(
            x_hbm.at[pl.ds(base + c * CHUNK, CHUNK)], xb[c % 2], si[c % 2])

    in_dma = {0: start_in(0)}
    out_dma = {}
    for c in range(NCH):
        b = c % 2
        in_dma.pop(c).wait()
        if c + 1 < NCH:
            in_dma[c + 1] = start_in(c + 1)
        if c - 2 in out_dma:
            out_dma.pop(c - 2).wait()
        compute_chunk(xb[b], ob[b])
        out_dma[c] = pltpu.async_copy(
            ob[b], out_hbm.at[pl.ds(base + c * CHUNK, CHUNK)], so[b])
    for c in sorted(out_dma):
        out_dma.pop(c).wait()


@functools.lru_cache(maxsize=None)
def _make_quant_kernel(fast: bool):
    @functools.partial(
        pl.kernel,
        out_type=(
            jax.ShapeDtypeStruct((N,), _f32),
            jax.ShapeDtypeStruct((NW, L), _f32),
            jax.ShapeDtypeStruct((NW, L), _f32),
        ),
        mesh=plsc.VectorSubcoreMesh(core_axis_name="c", subcore_axis_name="s",
                                    num_cores=NC, num_subcores=NS),
        compiler_params=pltpu.CompilerParams(needs_layout_passes=False),
        scratch_types=[
            pltpu.VMEM((CHUNK,), _f32),
            pltpu.VMEM((CHUNK,), _f32),
            pltpu.VMEM((CHUNK,), _f32),
            pltpu.VMEM((CHUNK,), _f32),
            pltpu.VMEM((L,), _f32),
            pltpu.VMEM((L,), _f32),
            pltpu.VMEM((L,), _f32),
            pltpu.VMEM((L,), _f32),
            pltpu.SemaphoreType.DMA,
            pltpu.SemaphoreType.DMA,
            pltpu.SemaphoreType.DMA,
            pltpu.SemaphoreType.DMA,
        ],
    )
    def quant(x_hbm, params_hbm, lut_hbm, q_hbm, mn_hbm, mx_hbm,
              xb0, xb1, ob0, ob1, pv, lutv, mnv, mxv, si0, si1, so0, so1):
        wid = _worker_id()
        pltpu.sync_copy(params_hbm, pv)
        pltpu.sync_copy(lut_hbm, lutv)

        pvec = pv[...]
        if fast:
            a = pvec[0]
            inv_h = pvec[1]
            m1i = pvec[4].astype(jnp.int32)
        else:
            mids = [pvec[k] for k in range(8)]

        state = [jnp.full((L,), jnp.inf, _f32), jnp.full((L,), -jnp.inf, _f32)]

        def compute_chunk(xb, ob):
            mn0, mx0 = state

            def vbody(i, carry):
                mn, mx = carry
                for u in range(U):
                    off = i * (U * L) + u * L
                    xv = xb[pl.ds(off, L)]
                    if fast:
                        y = (xv - a) * inv_h
                        t = y.astype(jnp.int32)
                        tf = t.astype(_f32)
                        t = jnp.where(tf < y, t + 1, t)
                        t = jnp.minimum(jnp.maximum(t, 0), m1i)
                    else:
                        t = jnp.zeros((L,), jnp.int32)
                        for m in mids:
                            t = t + jnp.where(xv > m, 1, 0)
                    ob[pl.ds(off, L)] = plsc.load_gather(lutv, [t])
                    mn = jnp.minimum(mn, xv)
                    mx = jnp.maximum(mx, xv)
                return mn, mx

            state[0], state[1] = lax.fori_loop(0, VECS // U, vbody, (mn0, mx0))

        _streaming_body(compute_chunk, x_hbm, q_hbm,
                        (xb0, xb1), (ob0, ob1), (si0, si1), (so0, so1))

        mnv[...] = state[0]
        mxv[...] = state[1]
        pltpu.sync_copy(mnv, mn_hbm.at[wid])
        pltpu.sync_copy(mxv, mx_hbm.at[wid])

    return quant


@functools.lru_cache(maxsize=None)
def _make_clip_kernel():
    @functools.partial(
        pl.kernel,
        out_type=jax.ShapeDtypeStruct((N,), _f32),
        mesh=plsc.VectorSubcoreMesh(core_axis_name="c", subcore_axis_name="s",
                                    num_cores=NC, num_subcores=NS),
        compiler_params=pltpu.CompilerParams(needs_layout_passes=False),
        scratch_types=[
            pltpu.VMEM((CHUNK,), _f32),
            pltpu.VMEM((CHUNK,), _f32),
            pltpu.VMEM((CHUNK,), _f32),
            pltpu.VMEM((CHUNK,), _f32),
            pltpu.VMEM((L,), _f32),
            pltpu.SemaphoreType.DMA,
            pltpu.SemaphoreType.DMA,
            pltpu.SemaphoreType.DMA,
            pltpu.SemaphoreType.DMA,
        ],
    )
    def _clip_kernel(q_hbm, bounds_hbm, out_hbm,
                     xb0, xb1, ob0, ob1, bv, si0, si1, so0, so1):
        pltpu.sync_copy(bounds_hbm, bv)
        bvec = bv[...]
        lo = bvec[0]
        hi = bvec[1]

        def compute_chunk(xb, ob):
            def vbody(i, carry):
                for u in range(U):
                    off = i * (U * L) + u * L
                    v = xb[pl.ds(off, L)]
                    ob[pl.ds(off, L)] = jnp.minimum(jnp.maximum(v, lo), hi)
                return carry

            lax.fori_loop(0, VECS // U, vbody, 0)

        _streaming_body(compute_chunk, q_hbm, out_hbm,
                        (xb0, xb1), (ob0, ob1), (si0, si1), (so0, so1))

    return _clip_kernel


def kernel(x, scales):
    s0 = scales[0, 0]
    s1 = scales[1, 0]
    vals = jnp.array([-1.0, 0.0, 1.0], _f32)
    levels = (vals[:, None] * s0 + vals[None, :] * s1).ravel()
    slv = jnp.sort(levels)
    gaps = slv[1:] - slv[:-1]
    mids = 0.5 * (slv[1:] + slv[:-1])
    h = jnp.max(gaps)
    uniform = (h > 0) & jnp.all((gaps == 0) | (gaps == h))
    m_minus_1 = jnp.round((slv[8] - slv[0]) / jnp.where(h > 0, h, 1.0))
    a = slv[0] + 0.5 * h

    pad = jnp.zeros((L - 8,), _f32)
    params_fast = jnp.concatenate(
        [jnp.stack([a, 1.0 / jnp.where(h > 0, h, 1.0), h, slv[0], m_minus_1]),
         jnp.zeros((L - 5,), _f32)])
    params_gen = jnp.concatenate([mids, pad])
    lut_fast = slv[0] + jnp.arange(L, dtype=_f32) * h
    lut_gen = jnp.concatenate([slv, jnp.full((L - 9,), slv[8], _f32)])

    xf = x.reshape(N)
    quant_fast = _make_quant_kernel(True)
    quant_generic = _make_quant_kernel(False)
    q, mn, mx = lax.cond(
        uniform,
        lambda o, pf, lf, pg, lg: quant_fast(o, pf, lf),
        lambda o, pf, lf, pg, lg: quant_generic(o, pg, lg),
        xf, params_fast, lut_fast, params_gen, lut_gen)

    xmin = jnp.min(mn)
    xmax = jnp.max(mx)
    need_clip = (xmin > slv[0]) | (xmax < slv[8])
    bounds = jnp.concatenate([jnp.stack([xmin, xmax]), jnp.zeros((L - 2,), _f32)])
    clip_kernel = _make_clip_kernel()
    out = lax.cond(need_clip,
                   lambda qq, bb: clip_kernel(qq, bb),
                   lambda qq, bb: qq,
                   q, bounds)
    return out.reshape(x.shape)


# speedup vs baseline: 3.2380x; 1.1142x over previous
import functools

import jax
import jax.numpy as jnp
from jax import lax
from jax.experimental import pallas as pl
from jax.experimental.pallas import tpu as pltpu
from jax.experimental.pallas import tpu_sc as plsc

NC = 2
NS = 16
NW = NC * NS
L = 16

R, C = 2048, 2048
ROWS_W = R // NW
ROWS_CH = 8
NCH = ROWS_W // ROWS_CH
CVEC = C // L

BIAS = 12582912.0
IBIAS = 12582911

_f32 = jnp.float32


def _worker_id():
    return lax.axis_index("c") * NS + lax.axis_index("s")


def _stream(compute_chunk, x_hbm, out_hbm, xb, ob, si, so):
    row0 = _worker_id() * ROWS_W

    def start_in(c):
        return pltpu.async_copy(
            x_hbm.at[pl.ds(row0 + c * ROWS_CH, ROWS_CH), :], xb[c % 2],
            si[c % 2])

    in_dma = {0: start_in(0)}
    out_dma = {}
    for c in range(NCH):
        b = c % 2
        in_dma.pop(c).wait()
        if c + 1 < NCH:
            in_dma[c + 1] = start_in(c + 1)
        if c - 2 in out_dma:
            out_dma.pop(c - 2).wait()
        compute_chunk(xb[b], None if ob is None else ob[b])
        if ob is not None:
            out_dma[c] = pltpu.async_copy(
                ob[b], out_hbm.at[pl.ds(row0 + c * ROWS_CH, ROWS_CH), :],
                so[b])
    for c in sorted(out_dma):
        out_dma.pop(c).wait()


@functools.lru_cache(maxsize=None)
def _make_minmax_kernel():
    @functools.partial(
        pl.kernel,
        out_type=(
            jax.ShapeDtypeStruct((NW, L), _f32),
            jax.ShapeDtypeStruct((NW, L), _f32),
        ),
        mesh=plsc.VectorSubcoreMesh(core_axis_name="c", subcore_axis_name="s",
                                    num_cores=NC, num_subcores=NS),
        compiler_params=pltpu.CompilerParams(needs_layout_passes=False),
        scratch_types=[
            pltpu.VMEM((ROWS_CH, C), _f32),
            pltpu.VMEM((ROWS_CH, C), _f32),
            pltpu.VMEM((L,), _f32),
            pltpu.VMEM((L,), _f32),
            pltpu.SemaphoreType.DMA,
            pltpu.SemaphoreType.DMA,
        ],
    )
    def minmax(x_hbm, mn_hbm, mx_hbm, xb0, xb1, mnv, mxv, si0, si1):
        wid = _worker_id()
        state = [jnp.full((L,), jnp.inf, _f32), jnp.full((L,), -jnp.inf, _f32)]

        def compute_chunk(xb, _):
            def vbody(i, carry):
                mn, mx = carry
                for r in range(ROWS_CH):
                    xv = xb[r, pl.ds(i * L, L)]
                    mn = jnp.minimum(mn, xv)
                    mx = jnp.maximum(mx, xv)
                return mn, mx

            state[0], state[1] = lax.fori_loop(0, CVEC, vbody,
                                               (state[0], state[1]))

        _stream(compute_chunk, x_hbm, None, (xb0, xb1), None, (si0, si1),
                None)
        mnv[...] = state[0]
        mxv[...] = state[1]
        pltpu.sync_copy(mnv, mn_hbm.at[wid])
        pltpu.sync_copy(mxv, mx_hbm.at[wid])

    return minmax


@functools.lru_cache(maxsize=None)
def _make_quant_kernel():
    @functools.partial(
        pl.kernel,
        out_type=jax.ShapeDtypeStruct((R, C), _f32),
        mesh=plsc.VectorSubcoreMesh(core_axis_name="c", subcore_axis_name="s",
                                    num_cores=NC, num_subcores=NS),
        compiler_params=pltpu.CompilerParams(needs_layout_passes=False),
        scratch_types=[
            pltpu.VMEM((ROWS_CH, C), _f32),
            pltpu.VMEM((ROWS_CH, C), _f32),
            pltpu.VMEM((ROWS_CH, C), _f32),
            pltpu.VMEM((ROWS_CH, C), _f32),
            pltpu.VMEM((L,), _f32),
            pltpu.VMEM((L,), _f32),
            pltpu.VMEM((NW, L), _f32),
            pltpu.VMEM((NW, L), _f32),
            pltpu.SemaphoreType.DMA,
            pltpu.SemaphoreType.DMA,
            pltpu.SemaphoreType.DMA,
            pltpu.SemaphoreType.DMA,
        ],
    )
    def quant(x_hbm, params_hbm, slv_hbm, mn_hbm, mx_hbm, out_hbm,
              xb0, xb1, ob0, ob1, pv, lutv, pmn, pmx, si0, si1, so0, so1):
        pltpu.sync_copy(params_hbm, pv)
        pltpu.sync_copy(mn_hbm, pmn)
        pltpu.sync_copy(mx_hbm, pmx)

        mnvec = pmn[0, pl.ds(0, L)]
        mxvec = pmx[0, pl.ds(0, L)]
        for w in range(1, NW):
            mnvec = jnp.minimum(mnvec, pmn[w, pl.ds(0, L)])
            mxvec = jnp.maximum(mxvec, pmx[w, pl.ds(0, L)])
        xmin = jnp.min(mnvec)
        xmax = jnp.max(mxvec)

        pvec = pv[...]
        is_fast = pvec[0] > 0.5
        inv_h = pvec[1]
        b0 = pvec[2]
        m1i = pvec[3].astype(jnp.int32)
        h = pvec[4]
        lvl0 = pvec[5]
        mids = [pvec[7 + k] for k in range(8)]

        grid = lvl0 + jnp.arange(L, dtype=jnp.int32).astype(_f32) * h
        lutv[...] = jnp.minimum(jnp.maximum(grid, xmin), xmax)

        @pl.when(jnp.logical_not(is_fast))
        def _():
            pltpu.sync_copy(slv_hbm, lutv)
            lutv[...] = jnp.minimum(jnp.maximum(lutv[...], xmin), xmax)

        def fast_chunk(xb, ob):
            def vbody(i, _):
                for r in range(ROWS_CH):
                    xv = xb[r, pl.ds(i * L, L)]
                    tg = (xv * inv_h + b0) + BIAS
                    t = tg.astype(jnp.int32) - IBIAS
                    t = jnp.minimum(jnp.maximum(t, 0), m1i)
                    ob[r, pl.ds(i * L, L)] = plsc.load_gather(lutv, [t])
                return 0

            lax.fori_loop(0, CVEC, vbody, 0)

        def gen_chunk(xb, ob):
            def vbody(i, _):
                for r in range(ROWS_CH):
                    xv = xb[r, pl.ds(i * L, L)]
                    t = jnp.zeros((L,), jnp.int32)
                    for m in mids:
                        t = t + jnp.where(xv > m, 1, 0)
                    ob[r, pl.ds(i * L, L)] = plsc.load_gather(lutv, [t])
                return 0

            lax.fori_loop(0, CVEC, vbody, 0)

        @pl.when(is_fast)
        def _():
            _stream(fast_chunk, x_hbm, out_hbm, (xb0, xb1), (ob0, ob1),
                    (si0, si1), (so0, so1))

        @pl.when(jnp.logical_not(is_fast))
        def _():
            _stream(gen_chunk, x_hbm, out_hbm, (xb0, xb1), (ob0, ob1),
                    (si0, si1), (so0, so1))

    return quant


def kernel(x, scales):
    s0 = scales[0, 0]
    s1 = scales[1, 0]
    vals = jnp.array([-1.0, 0.0, 1.0], _f32)
    levels = (vals[:, None] * s0 + vals[None, :] * s1).ravel()
    slv = jnp.sort(levels)
    gaps = slv[1:] - slv[:-1]
    mids = 0.5 * (slv[1:] + slv[:-1])
    h = jnp.max(gaps)
    uniform = (h > 0) & jnp.all((gaps == 0) | (gaps == h))
    hs = jnp.where(h > 0, h, 1.0)
    inv_h = 1.0 / hs
    m_minus_1 = jnp.round((slv[8] - slv[0]) / hs)
    a = slv[0] + 0.5 * h
    b0 = -a * inv_h - 0.5
    flag = jnp.where(uniform, 1.0, 0.0).astype(_f32)

    params = jnp.concatenate([
        jnp.stack([flag, inv_h, b0, m_minus_1, h, slv[0], slv[8]]),
        mids, jnp.zeros((1,), _f32)])
    slv16 = jnp.concatenate([slv, jnp.full((L - 9,), slv[8], _f32)])

    mn, mx = _make_minmax_kernel()(x)
    out = _make_quant_kernel()(x, params, slv16, mn, mx)
    return out


# speedup vs baseline: 3.5991x; 1.1115x over previous
import functools

import jax
import jax.numpy as jnp
from jax import lax
from jax.experimental import pallas as pl
from jax.experimental.pallas import tpu as pltpu
from jax.experimental.pallas import tpu_sc as plsc

NC = 2
NS = 16
NW = NC * NS
L = 16

R, C = 2048, 2048
ROWS_W = R // NW
ROWS_CH = 8
NCH = ROWS_W // ROWS_CH
CVEC = C // L

NBUCKET = 2048
BVEC = NBUCKET // L

_f32 = jnp.float32
_i32 = jnp.int32


def _worker_id():
    return lax.axis_index("c") * NS + lax.axis_index("s")


def _stream(compute_chunk, x_hbm, out_hbm, xb, ob, si, so, prelude=None):
    row0 = _worker_id() * ROWS_W

    def start_in(c):
        return pltpu.async_copy(
            x_hbm.at[pl.ds(row0 + c * ROWS_CH, ROWS_CH), :], xb[c % 2],
            si[c % 2])

    in_dma = {0: start_in(0)}
    if prelude is not None:
        prelude()
    out_dma = {}
    for c in range(NCH):
        b = c % 2
        in_dma.pop(c).wait()
        if c + 1 < NCH:
            in_dma[c + 1] = start_in(c + 1)
        if c - 2 in out_dma:
            out_dma.pop(c - 2).wait()
        compute_chunk(xb[b], None if ob is None else ob[b])
        if ob is not None:
            out_dma[c] = pltpu.async_copy(
                ob[b], out_hbm.at[pl.ds(row0 + c * ROWS_CH, ROWS_CH), :],
                so[b])
    for c in sorted(out_dma):
        out_dma.pop(c).wait()


@functools.lru_cache(maxsize=None)
def _make_minmax_kernel():
    @functools.partial(
        pl.kernel,
        out_type=(
            jax.ShapeDtypeStruct((NW, L), _f32),
            jax.ShapeDtypeStruct((NW, L), _f32),
        ),
        mesh=plsc.VectorSubcoreMesh(core_axis_name="c", subcore_axis_name="s",
                                    num_cores=NC, num_subcores=NS),
        compiler_params=pltpu.CompilerParams(needs_layout_passes=False),
        scratch_types=[
            pltpu.VMEM((ROWS_CH, C), _f32),
            pltpu.VMEM((ROWS_CH, C), _f32),
            pltpu.VMEM((L,), _f32),
            pltpu.VMEM((L,), _f32),
            pltpu.SemaphoreType.DMA,
            pltpu.SemaphoreType.DMA,
        ],
    )
    def minmax(x_hbm, mn_hbm, mx_hbm, xb0, xb1, mnv, mxv, si0, si1):
        wid = _worker_id()
        state = [jnp.full((L,), jnp.inf, _f32), jnp.full((L,), -jnp.inf, _f32)]

        def compute_chunk(xb, _):
            def vbody(i, carry):
                mn, mx = carry
                for r in range(ROWS_CH):
                    xv = xb[r, pl.ds(i * L, L)]
                    mn = jnp.minimum(mn, xv)
                    mx = jnp.maximum(mx, xv)
                return mn, mx

            state[0], state[1] = lax.fori_loop(0, CVEC, vbody,
                                               (state[0], state[1]))

        _stream(compute_chunk, x_hbm, None, (xb0, xb1), None, (si0, si1),
                None)
        mnv[...] = state[0]
        mxv[...] = state[1]
        pltpu.sync_copy(mnv, mn_hbm.at[wid])
        pltpu.sync_copy(mxv, mx_hbm.at[wid])

    return minmax


@functools.lru_cache(maxsize=None)
def _make_quant_kernel():
    @functools.partial(
        pl.kernel,
        out_type=jax.ShapeDtypeStruct((R, C), _f32),
        mesh=plsc.VectorSubcoreMesh(core_axis_name="c", subcore_axis_name="s",
                                    num_cores=NC, num_subcores=NS),
        compiler_params=pltpu.CompilerParams(needs_layout_passes=False),
        scratch_types=[
            pltpu.VMEM((ROWS_CH, C), _f32),
            pltpu.VMEM((ROWS_CH, C), _f32),
            pltpu.VMEM((ROWS_CH, C), _f32),
            pltpu.VMEM((ROWS_CH, C), _f32),
            pltpu.VMEM((NBUCKET,), _f32),
            pltpu.VMEM((L,), _f32),
            pltpu.VMEM((L,), _f32),
            pltpu.VMEM((NW, L), _f32),
            pltpu.VMEM((NW, L), _f32),
            pltpu.SemaphoreType.DMA,
            pltpu.SemaphoreType.DMA,
            pltpu.SemaphoreType.DMA,
            pltpu.SemaphoreType.DMA,
        ],
    )
    def quant(x_hbm, params_hbm, slv_hbm, mn_hbm, mx_hbm, out_hbm,
              xb0, xb1, ob0, ob1, lutv, pv, slvv, pmn, pmx,
              si0, si1, so0, so1):
        pltpu.sync_copy(params_hbm, pv)
        pltpu.sync_copy(slv_hbm, slvv)
        pvec = pv[...]
        aligned = pvec[0] > 0.5
        mids = [pvec[1 + k] for k in range(8)]

        def bounds():
            pltpu.sync_copy(mn_hbm, pmn)
            pltpu.sync_copy(mx_hbm, pmx)
            mnvec = pmn[0, pl.ds(0, L)]
            mxvec = pmx[0, pl.ds(0, L)]
            for w in range(1, NW):
                mnvec = jnp.minimum(mnvec, pmn[w, pl.ds(0, L)])
                mxvec = jnp.maximum(mxvec, pmx[w, pl.ds(0, L)])
            return jnp.min(mnvec), jnp.max(mxvec)

        def classify(xv, xmin, xmax):
            t = jnp.zeros((L,), _i32)
            for m in mids:
                t = t + jnp.where(xv > m, 1, 0)
            q = plsc.load_gather(slvv, [t])
            return jnp.minimum(jnp.maximum(q, xmin), xmax)

        def prep_bucket():
            xmin, xmax = bounds()
            base = jax.lax.iota(_i32, L)

            def bbody(i, _):
                bits = (i * L + base) << 21 | 0x100000
                rep = plsc.bitcast(bits, _f32)
                lutv[pl.ds(i * L, L)] = classify(rep, xmin, xmax)
                return 0

            lax.fori_loop(0, BVEC, bbody, 0)

        def bucket_chunk(xb, ob):
            def vbody(i, _):
                for r in range(ROWS_CH):
                    xv = xb[r, pl.ds(i * L, L)]
                    t = jnp.right_shift(plsc.bitcast(xv, _i32), 21) & 0x7FF
                    ob[r, pl.ds(i * L, L)] = plsc.load_gather(lutv, [t])
                return 0

            lax.fori_loop(0, CVEC, vbody, 0)

        gen_state = {}

        def prep_gen():
            gen_state["b"] = bounds()

        def gen_chunk(xb, ob):
            xmin, xmax = gen_state["b"]

            def vbody(i, _):
                for r in range(ROWS_CH):
                    xv = xb[r, pl.ds(i * L, L)]
                    ob[r, pl.ds(i * L, L)] = classify(xv, xmin, xmax)
                return 0

            lax.fori_loop(0, CVEC, vbody, 0)

        @pl.when(aligned)
        def _():
            _stream(bucket_chunk, x_hbm, out_hbm, (xb0, xb1), (ob0, ob1),
                    (si0, si1), (so0, so1), prelude=prep_bucket)

        @pl.when(jnp.logical_not(aligned))
        def _():
            _stream(gen_chunk, x_hbm, out_hbm, (xb0, xb1), (ob0, ob1),
                    (si0, si1), (so0, so1), prelude=prep_gen)

    return quant


def kernel(x, scales):
    s0 = scales[0, 0]
    s1 = scales[1, 0]
    vals = jnp.array([-1.0, 0.0, 1.0], _f32)
    levels = (vals[:, None] * s0 + vals[None, :] * s1).ravel()
    slv = jnp.sort(levels)
    mids = 0.5 * (slv[1:] + slv[:-1])
    mbits = jax.lax.bitcast_convert_type(mids, _i32)
    aligned = jnp.all((mbits & 0x1FFFFF) == 0)
    flag = jnp.where(aligned, 1.0, 0.0).astype(_f32)

    params = jnp.concatenate([flag[None], mids, jnp.zeros((7,), _f32)])
    slv16 = jnp.concatenate([slv, jnp.full((L - 9,), slv[8], _f32)])

    mn, mx = _make_minmax_kernel()(x)
    out = _make_quant_kernel()(x, params, slv16, mn, mx)
    return out


# speedup vs baseline: 6.1636x; 1.7126x over previous
import functools

import jax
import jax.numpy as jnp
from jax import lax
from jax.experimental import pallas as pl
from jax.experimental.pallas import tpu as pltpu
from jax.experimental.pallas import tpu_sc as plsc

NC = 2
NS = 16
NW = NC * NS
L = 16

R, C = 2048, 2048
ROWS_W = R // NW
ROWS_CH = 8
NCH = ROWS_W // ROWS_CH
CVEC = C // L

NBUCKET = 2048
BVEC = NBUCKET // L

_f32 = jnp.float32
_i32 = jnp.int32


def _worker_id():
    return lax.axis_index("c") * NS + lax.axis_index("s")


@functools.lru_cache(maxsize=None)
def _make_minmax_kernel():
    @functools.partial(
        pl.kernel,
        out_type=(
            jax.ShapeDtypeStruct((NW, L), _f32),
            jax.ShapeDtypeStruct((NW, L), _f32),
        ),
        mesh=plsc.VectorSubcoreMesh(core_axis_name="c", subcore_axis_name="s",
                                    num_cores=NC, num_subcores=NS),
        compiler_params=pltpu.CompilerParams(needs_layout_passes=False),
        scratch_types=[
            pltpu.VMEM((ROWS_CH, C), _f32),
            pltpu.VMEM((ROWS_CH, C), _f32),
            pltpu.VMEM((L,), _f32),
            pltpu.VMEM((L,), _f32),
            pltpu.SemaphoreType.DMA,
            pltpu.SemaphoreType.DMA,
        ],
    )
    def minmax(x_hbm, mn_hbm, mx_hbm, xb0, xb1, mnv, mxv, si0, si1):
        wid = _worker_id()
        row0 = wid * ROWS_W
        xb = (xb0, xb1)
        si = (si0, si1)

        def start_in(ch, b):
            return pltpu.async_copy(
                x_hbm.at[pl.ds(row0 + ch * ROWS_CH, ROWS_CH), :], xb[b],
                si[b])

        start_in(0, 0)
        start_in(1, 1)

        def pair(k, carry):
            for b in range(2):
                ch = 2 * k + b
                pltpu.make_async_copy(
                    x_hbm.at[pl.ds(row0, ROWS_CH), :], xb[b], si[b]).wait()

                def vbody(i, c2):
                    mn, mx = c2
                    for r in range(ROWS_CH):
                        xv = xb[b][r, pl.ds(i * L, L)]
                        mn = jnp.minimum(mn, xv)
                        mx = jnp.maximum(mx, xv)
                    return mn, mx

                carry = lax.fori_loop(0, CVEC, vbody, carry)

                @pl.when(ch + 2 < NCH)
                def _():
                    start_in(ch + 2, b)

            return carry

        init = (jnp.full((L,), jnp.inf, _f32), jnp.full((L,), -jnp.inf, _f32))
        mn, mx = lax.fori_loop(0, NCH // 2, pair, init)
        mnv[...] = mn
        mxv[...] = mx
        pltpu.sync_copy(mnv, mn_hbm.at[wid])
        pltpu.sync_copy(mxv, mx_hbm.at[wid])

    return minmax


@functools.lru_cache(maxsize=None)
def _make_quant_kernel():
    @functools.partial(
        pl.kernel,
        out_type=jax.ShapeDtypeStruct((R, C), _f32),
        mesh=plsc.VectorSubcoreMesh(core_axis_name="c", subcore_axis_name="s",
                                    num_cores=NC, num_subcores=NS),
        compiler_params=pltpu.CompilerParams(needs_layout_passes=False),
        scratch_types=[
            pltpu.VMEM((ROWS_CH, C), _f32),
            pltpu.VMEM((ROWS_CH, C), _f32),
            pltpu.VMEM((ROWS_CH, C), _f32),
            pltpu.VMEM((ROWS_CH, C), _f32),
            pltpu.VMEM((NBUCKET,), _f32),
            pltpu.VMEM((L,), _f32),
            pltpu.VMEM((L,), _f32),
            pltpu.VMEM((NW, L), _f32),
            pltpu.VMEM((NW, L), _f32),
            pltpu.SemaphoreType.DMA,
            pltpu.SemaphoreType.DMA,
            pltpu.SemaphoreType.DMA,
            pltpu.SemaphoreType.DMA,
        ],
    )
    def quant(x_hbm, params_hbm, slv_hbm, mn_hbm, mx_hbm, out_hbm,
              xb0, xb1, ob0, ob1, lutv, pv, slvv, pmn, pmx,
              si0, si1, so0, so1):
        row0 = _worker_id() * ROWS_W
        xb = (xb0, xb1)
        ob = (ob0, ob1)
        si = (si0, si1)
        so = (so0, so1)

        pltpu.sync_copy(params_hbm, pv)
        pltpu.sync_copy(slv_hbm, slvv)
        pvec = pv[...]
        aligned = pvec[0] > 0.5
        mids = [pvec[1 + k] for k in range(8)]

        def bounds():
            pltpu.sync_copy(mn_hbm, pmn)
            pltpu.sync_copy(mx_hbm, pmx)
            mnvec = pmn[0, pl.ds(0, L)]
            mxvec = pmx[0, pl.ds(0, L)]
            for w in range(1, NW):
                mnvec = jnp.minimum(mnvec, pmn[w, pl.ds(0, L)])
                mxvec = jnp.maximum(mxvec, pmx[w, pl.ds(0, L)])
            return jnp.min(mnvec), jnp.max(mxvec)

        def classify(xv, xmin, xmax):
            t = jnp.zeros((L,), _i32)
            for m in mids:
                t = t + jnp.where(xv > m, 1, 0)
            q = plsc.load_gather(slvv, [t])
            return jnp.minimum(jnp.maximum(q, xmin), xmax)

        def stream(compute_chunk, prelude):
            def start_in(ch, b):
                return pltpu.async_copy(
                    x_hbm.at[pl.ds(row0 + ch * ROWS_CH, ROWS_CH), :], xb[b],
                    si[b])

            start_in(0, 0)
            prelude()
            start_in(1, 1)

            def pair(k, _):
                for b in range(2):
                    ch = 2 * k + b
                    pltpu.make_async_copy(
                        x_hbm.at[pl.ds(row0, ROWS_CH), :], xb[b],
                        si[b]).wait()

                    @pl.when(ch >= 2)
                    def _():
                        pltpu.make_async_copy(
                            ob[b], out_hbm.at[pl.ds(row0, ROWS_CH), :],
                            so[b]).wait()

                    compute_chunk(xb[b], ob[b])

                    @pl.when(ch + 2 < NCH)
                    def _():
                        start_in(ch + 2, b)

                    pltpu.async_copy(
                        ob[b],
                        out_hbm.at[pl.ds(row0 + ch * ROWS_CH, ROWS_CH), :],
                        so[b])
                return 0

            lax.fori_loop(0, NCH // 2, pair, 0)
            for b in range(2):
                pltpu.make_async_copy(
                    ob[b], out_hbm.at[pl.ds(row0, ROWS_CH), :], so[b]).wait()

        def prep_bucket():
            xmin, xmax = bounds()
            base = lax.iota(_i32, L)

            @plsc.parallel_loop(0, BVEC)
            def _(i):
                bits = ((i * L + base) << 21) | 0x100000
                rep = plsc.bitcast(bits, _f32)
                lutv[pl.ds(i * L, L)] = classify(rep, xmin, xmax)

        def bucket_chunk(cxb, cob):
            @plsc.parallel_loop(0, CVEC, unroll=2)
            def _(i):
                for r in range(ROWS_CH):
                    xv = cxb[r, pl.ds(i * L, L)]
                    t = jnp.right_shift(plsc.bitcast(xv, _i32), 21) & 0x7FF
                    cob[r, pl.ds(i * L, L)] = plsc.load_gather(lutv, [t])

        gen_state = {}

        def prep_gen():
            gen_state["b"] = bounds()

        def gen_chunk(cxb, cob):
            xmin, xmax = gen_state["b"]

            @plsc.parallel_loop(0, CVEC)
            def _(i):
                for r in range(ROWS_CH):
                    xv = cxb[r, pl.ds(i * L, L)]
                    cob[r, pl.ds(i * L, L)] = classify(xv, xmin, xmax)

        @pl.when(aligned)
        def _():
            stream(bucket_chunk, prep_bucket)

        @pl.when(jnp.logical_not(aligned))
        def _():
            stream(gen_chunk, prep_gen)

    return quant


def kernel(x, scales):
    s0 = scales[0, 0]
    s1 = scales[1, 0]
    vals = jnp.array([-1.0, 0.0, 1.0], _f32)
    levels = (vals[:, None] * s0 + vals[None, :] * s1).ravel()
    slv = jnp.sort(levels)
    mids = 0.5 * (slv[1:] + slv[:-1])
    mbits = jax.lax.bitcast_convert_type(mids, _i32)
    aligned = jnp.all((mbits & 0x1FFFFF) == 0)
    flag = jnp.where(aligned, 1.0, 0.0).astype(_f32)

    params = jnp.concatenate([flag[None], mids, jnp.zeros((7,), _f32)])
    slv16 = jnp.concatenate([slv, jnp.full((L - 9,), slv[8], _f32)])

    mn, mx = _make_minmax_kernel()(x)
    out = _make_quant_kernel()(x, params, slv16, mn, mx)
    return out
